# baseline (device time: 19119 ns/iter reference)
import jax
import jax.numpy as jnp
from jax import lax
from jax.experimental import pallas as pl
from jax.experimental.pallas import tpu as pltpu

N_CHUNKS = 4


def kernel(x, pi):
    _, m, n = x.shape
    chunk = m // N_CHUNKS

    def body(x_ref, pi_ref, out_ref, stage, copy_sems, send_sems, recv_sems):
        my_pos = lax.axis_index("i")
        dst = pi_ref[my_pos]
        src = jnp.int32(0)
        for j in range(4):
            src = jnp.where(pi_ref[j] == my_pos, jnp.int32(j), src)

        copies = []
        for c in range(N_CHUNKS):
            cp = pltpu.make_async_copy(
                x_ref.at[0, pl.ds(c * chunk, chunk), :],
                stage.at[c],
                copy_sems.at[c],
            )
            cp.start()
            copies.append(cp)

        barrier_sem = pltpu.get_barrier_semaphore()
        pl.semaphore_signal(
            barrier_sem, inc=1,
            device_id=src, device_id_type=pl.DeviceIdType.LOGICAL,
        )
        pl.semaphore_wait(barrier_sem, 1)

        rdmas = []
        for c in range(N_CHUNKS):
            copies[c].wait()
            rdma = pltpu.make_async_remote_copy(
                src_ref=stage.at[c],
                dst_ref=out_ref.at[0, pl.ds(c * chunk, chunk), :],
                send_sem=send_sems.at[c],
                recv_sem=recv_sems.at[c],
                device_id=dst,
                device_id_type=pl.DeviceIdType.LOGICAL,
            )
            rdma.start()
            rdmas.append(rdma)
        for rdma in rdmas:
            rdma.wait_send()
        for rdma in rdmas:
            rdma.wait_recv()

    return pl.pallas_call(
        body,
        out_shape=jax.ShapeDtypeStruct(x.shape, x.dtype),
        in_specs=[
            pl.BlockSpec(memory_space=pl.ANY),
            pl.BlockSpec(memory_space=pltpu.SMEM),
        ],
        out_specs=pl.BlockSpec(memory_space=pl.ANY),
        scratch_shapes=[
            pltpu.VMEM((N_CHUNKS, chunk, n), x.dtype),
            pltpu.SemaphoreType.DMA((N_CHUNKS,)),
            pltpu.SemaphoreType.DMA((N_CHUNKS,)),
            pltpu.SemaphoreType.DMA((N_CHUNKS,)),
        ],
        compiler_params=pltpu.CompilerParams(collective_id=0),
    )(x, pi)
